# baseline (speedup 1.0000x reference)
import functools

import jax
import jax.numpy as jnp
from jax import lax
from jax.experimental import pallas as pl
from jax.experimental.pallas import tpu as pltpu
from jax.experimental.pallas import tpu_sc as plsc

NUM_CORES = 2
NUM_SUBCORES = 16
NUM_WORKERS = NUM_CORES * NUM_SUBCORES
LANES = 16

BATCH = 4096
HIST = 200
DIM = 64
CHUNK = 128
SCALE = 8.0
NBUF = 4
DSUB = DIM // 8

_mesh = plsc.VectorSubcoreMesh(core_axis_name="c", subcore_axis_name="s")


@functools.partial(
    pl.kernel,
    out_type=jax.ShapeDtypeStruct((HIST, 8, NUM_WORKERS, 8, CHUNK),
                                  jnp.float32),
    mesh=_mesh,
    scratch_types=[
        pltpu.VMEM((HIST, CHUNK), jnp.int32),
        pltpu.VMEM((NBUF, CHUNK, DIM), jnp.float32),
        pltpu.VMEM((NBUF, 8, 8, CHUNK), jnp.float32),
        pltpu.SemaphoreType.DMA((NBUF,)),
        pltpu.SemaphoreType.DMA((NBUF,)),
    ],
    compiler_params=pltpu.CompilerParams(use_tc_tiling_on_sc=False,
                                         needs_layout_passes=False),
)
def _embed_sc(idx_hbm, table_hbm, out_hbm, idx_v, in_v, out_v, gsem, ssem):
    wid = lax.axis_index("s") * NUM_CORES + lax.axis_index("c")
    pltpu.sync_copy(idx_hbm.at[wid], idx_v)
    lane = lax.iota(jnp.int32, LANES)

    def start_gather(b, h):
        pltpu.async_copy(table_hbm.at[idx_v.at[h]], in_v.at[b], gsem.at[b])

    def wait_gather(b, h):
        pltpu.make_async_copy(table_hbm.at[idx_v.at[h]], in_v.at[b],
                              gsem.at[b]).wait()

    def start_store(b, h):
        for dt in range(8):
            pltpu.async_copy(out_v.at[b, dt], out_hbm.at[h, dt, wid],
                             ssem.at[b])

    def wait_store(b, h):
        for dt in range(8):
            pltpu.make_async_copy(out_v.at[b, dt], out_hbm.at[h, dt, wid],
                                  ssem.at[b]).wait()

    def transpose_scale(b):
        def d_body(d, carry):
            dt = d // 8
            dr = d % 8
            cols = jnp.full((LANES,), d, jnp.int32)
            for brc in range(CHUNK // LANES):
                rows = lane + (brc * LANES)
                v = plsc.load_gather(in_v.at[b], [rows, cols])
                out_v[b, dt, dr, pl.ds(brc * LANES, LANES)] = v * SCALE
            return carry

        lax.fori_loop(0, DIM, d_body, 0)

    for b in range(NBUF):
        start_gather(b, b)

    n_blocks = HIST // NBUF

    def block_body(h0, carry):
        for b in range(NBUF):
            h = h0 * NBUF + b
            wait_gather(b, h)

            @pl.when(h0 > 0)
            def _():
                wait_store(b, h - NBUF)

            transpose_scale(b)

            @pl.when(h0 < n_blocks - 1)
            def _():
                start_gather(b, h + NBUF)

            start_store(b, h)
        return carry

    lax.fori_loop(0, n_blocks, block_body, 0)

    for b in range(NBUF):
        wait_store(b, HIST - NBUF + b)


def kernel(x, embedding):
    xt = (x.astype(jnp.int32).T
          .reshape(HIST, NUM_WORKERS, CHUNK)
          .transpose(1, 0, 2))
    r = _embed_sc(xt, embedding)
    return r.transpose(2, 4, 0, 1, 3).reshape(BATCH, HIST, DIM)


# speedup vs baseline: 1.2138x; 1.2138x over previous
import functools

import jax
import jax.numpy as jnp
from jax import lax
from jax.experimental import pallas as pl
from jax.experimental.pallas import tpu as pltpu
from jax.experimental.pallas import tpu_sc as plsc

NUM_CORES = 2
NUM_SUBCORES = 16
NUM_WORKERS = NUM_CORES * NUM_SUBCORES
LANES = 16

BATCH = 4096
HIST = 200
DIM = 64
CHUNK = 128
SCALE = 8.0
NBUF = 4
DSUB = DIM // 8

_mesh = plsc.VectorSubcoreMesh(core_axis_name="c", subcore_axis_name="s")


@functools.partial(
    pl.kernel,
    out_type=jax.ShapeDtypeStruct((HIST, 8, NUM_WORKERS, 8, CHUNK),
                                  jnp.float32),
    mesh=_mesh,
    scratch_types=[
        pltpu.VMEM((HIST, CHUNK), jnp.int32),
        pltpu.VMEM((NBUF, CHUNK, DIM), jnp.float32),
        pltpu.VMEM((NBUF, DIM, CHUNK), jnp.float32),
        pltpu.SemaphoreType.DMA((NBUF,)),
        pltpu.SemaphoreType.DMA((NBUF,)),
    ],
    compiler_params=pltpu.CompilerParams(use_tc_tiling_on_sc=False,
                                         needs_layout_passes=False),
)
def _embed_sc(idx_hbm, table_hbm, out_hbm, idx_v, in_v, out_v, gsem, ssem):
    wid = lax.axis_index("s") * NUM_CORES + lax.axis_index("c")
    pltpu.sync_copy(idx_hbm.at[wid], idx_v)
    lane = lax.iota(jnp.int32, LANES)

    def start_gather(b, h):
        pltpu.async_copy(table_hbm.at[idx_v.at[h]], in_v.at[b], gsem.at[b])

    def wait_gather(b, h):
        pltpu.make_async_copy(table_hbm.at[idx_v.at[h]], in_v.at[b],
                              gsem.at[b]).wait()

    def start_store(b, h):
        for dt in range(8):
            pltpu.async_copy(out_v.at[b, pl.ds(dt * 8, 8)],
                             out_hbm.at[h, dt, wid], ssem.at[b])

    def wait_store(b, h):
        for dt in range(8):
            pltpu.make_async_copy(out_v.at[b, pl.ds(dt * 8, 8)],
                                  out_hbm.at[h, dt, wid],
                                  ssem.at[b]).wait()

    rows_list = [lane + (brc * LANES) for brc in range(CHUNK // LANES)]

    def transpose_scale(b):
        def d_body(d, carry):
            cols = jnp.full((LANES,), d, jnp.int32)
            vs = [plsc.load_gather(in_v.at[b], [rows, cols])
                  for rows in rows_list]
            for brc, v in enumerate(vs):
                out_v[b, d, pl.ds(brc * LANES, LANES)] = v * SCALE
            return carry

        lax.fori_loop(0, DIM, d_body, 0, unroll=2)

    for b in range(NBUF):
        start_gather(b, b)

    n_blocks = HIST // NBUF

    def block_body(h0, carry):
        for b in range(NBUF):
            h = h0 * NBUF + b
            wait_gather(b, h)

            @pl.when(h0 > 0)
            def _():
                wait_store(b, h - NBUF)

            transpose_scale(b)

            @pl.when(h0 < n_blocks - 1)
            def _():
                start_gather(b, h + NBUF)

            start_store(b, h)
        return carry

    lax.fori_loop(0, n_blocks, block_body, 0)

    for b in range(NBUF):
        wait_store(b, HIST - NBUF + b)


def kernel(x, embedding):
    xt = (x.astype(jnp.int32).T
          .reshape(HIST, NUM_WORKERS, CHUNK)
          .transpose(1, 0, 2))
    r = _embed_sc(xt, embedding)
    return r.transpose(2, 4, 0, 1, 3).reshape(BATCH, HIST, DIM)


# speedup vs baseline: 1.4633x; 1.2056x over previous
import functools

import jax
import jax.numpy as jnp
from jax import lax
from jax.experimental import pallas as pl
from jax.experimental.pallas import tpu as pltpu
from jax.experimental.pallas import tpu_sc as plsc

NUM_CORES = 2
NUM_SUBCORES = 16
NUM_WORKERS = NUM_CORES * NUM_SUBCORES
LANES = 16

BATCH = 4096
HIST = 200
DIM = 64
CHUNK = 128
SCALE = 8.0
NBUF = 4
DSUB = DIM // 8
PSTRIDE = 131

_mesh = plsc.VectorSubcoreMesh(core_axis_name="c", subcore_axis_name="s")


@functools.partial(
    pl.kernel,
    out_type=jax.ShapeDtypeStruct((HIST, 8, NUM_WORKERS, 8, CHUNK),
                                  jnp.float32),
    mesh=_mesh,
    scratch_types=[
        pltpu.VMEM((HIST, CHUNK), jnp.int32),
        pltpu.VMEM((NBUF, CHUNK, DIM), jnp.float32),
        pltpu.VMEM((DIM, PSTRIDE), jnp.float32),
        pltpu.VMEM((NBUF, DIM, CHUNK), jnp.float32),
        pltpu.SemaphoreType.DMA((NBUF,)),
        pltpu.SemaphoreType.DMA((NBUF,)),
    ],
    compiler_params=pltpu.CompilerParams(use_tc_tiling_on_sc=False,
                                         needs_layout_passes=False),
)
def _embed_sc(idx_hbm, table_hbm, out_hbm, idx_v, in_v, p_v, out_v, gsem, ssem):
    wid = lax.axis_index("s") * NUM_CORES + lax.axis_index("c")
    pltpu.sync_copy(idx_hbm.at[wid], idx_v)
    lane = lax.iota(jnp.int32, LANES)

    def start_gather(b, h):
        pltpu.async_copy(table_hbm.at[idx_v.at[h]], in_v.at[b], gsem.at[b])

    def wait_gather(b, h):
        pltpu.make_async_copy(table_hbm.at[idx_v.at[h]], in_v.at[b],
                              gsem.at[b]).wait()

    def start_store(b, h):
        for dt in range(8):
            pltpu.async_copy(out_v.at[b, pl.ds(dt * 8, 8)],
                             out_hbm.at[h, dt, wid], ssem.at[b])

    def wait_store(b, h):
        for dt in range(8):
            pltpu.make_async_copy(out_v.at[b, pl.ds(dt * 8, 8)],
                                  out_hbm.at[h, dt, wid],
                                  ssem.at[b]).wait()

    d_lists = [lane + (c * LANES) for c in range(DIM // LANES)]

    def transpose_scale(b):
        def br_body(br, carry):
            cols = jnp.full((LANES,), br, jnp.int32)
            vs = [in_v[b, br, pl.ds(c * LANES, LANES)]
                  for c in range(DIM // LANES)]
            for c, v in enumerate(vs):
                plsc.store_scatter(p_v, [d_lists[c], cols], v)
            return carry

        lax.fori_loop(0, CHUNK, br_body, 0, unroll=2)

        def d_body(d, carry):
            for c in range(CHUNK // LANES):
                sl = pl.ds(c * LANES, LANES)
                out_v[b, d, sl] = p_v[d, sl] * SCALE
            return carry

        lax.fori_loop(0, DIM, d_body, 0, unroll=2)

    for b in range(NBUF):
        start_gather(b, b)

    n_blocks = HIST // NBUF

    def block_body(h0, carry):
        for b in range(NBUF):
            h = h0 * NBUF + b
            wait_gather(b, h)

            @pl.when(h0 > 0)
            def _():
                wait_store(b, h - NBUF)

            transpose_scale(b)

            @pl.when(h0 < n_blocks - 1)
            def _():
                start_gather(b, h + NBUF)

            start_store(b, h)
        return carry

    lax.fori_loop(0, n_blocks, block_body, 0)

    for b in range(NBUF):
        wait_store(b, HIST - NBUF + b)


def kernel(x, embedding):
    xt = (x.astype(jnp.int32).T
          .reshape(HIST, NUM_WORKERS, CHUNK)
          .transpose(1, 0, 2))
    r = _embed_sc(xt, embedding)
    return r.transpose(2, 4, 0, 1, 3).reshape(BATCH, HIST, DIM)


# speedup vs baseline: 1.4694x; 1.0042x over previous
import functools

import jax
import jax.numpy as jnp
from jax import lax
from jax.experimental import pallas as pl
from jax.experimental.pallas import tpu as pltpu
from jax.experimental.pallas import tpu_sc as plsc

NUM_CORES = 2
NUM_SUBCORES = 16
NUM_WORKERS = NUM_CORES * NUM_SUBCORES
LANES = 16

BATCH = 4096
HIST = 200
DIM = 64
CHUNK = 128
SCALE = 8.0
NBUF = 4
DSUB = DIM // 8
PSTRIDE = 131

_mesh = plsc.VectorSubcoreMesh(core_axis_name="c", subcore_axis_name="s")


@functools.partial(
    pl.kernel,
    out_type=jax.ShapeDtypeStruct((HIST, 8, NUM_WORKERS, 8, CHUNK),
                                  jnp.float32),
    mesh=_mesh,
    scratch_types=[
        pltpu.VMEM((HIST, CHUNK), jnp.int32),
        pltpu.VMEM((NBUF, CHUNK, DIM), jnp.float32),
        pltpu.VMEM((DIM, PSTRIDE), jnp.float32),
        pltpu.VMEM((NBUF, 8, DSUB, CHUNK), jnp.float32),
        pltpu.SemaphoreType.DMA((NBUF,)),
        pltpu.SemaphoreType.DMA((NBUF,)),
    ],
    compiler_params=pltpu.CompilerParams(use_tc_tiling_on_sc=False,
                                         needs_layout_passes=False),
)
def _embed_sc(idx_hbm, table_hbm, out_hbm, idx_v, in_v, p_v, out_v, gsem, ssem):
    wid = lax.axis_index("s") * NUM_CORES + lax.axis_index("c")
    pltpu.sync_copy(idx_hbm.at[wid], idx_v)
    lane = lax.iota(jnp.int32, LANES)

    def start_gather(b, h):
        pltpu.async_copy(table_hbm.at[idx_v.at[h]], in_v.at[b], gsem.at[b])

    def wait_gather(b, h):
        pltpu.make_async_copy(table_hbm.at[idx_v.at[h]], in_v.at[b],
                              gsem.at[b]).wait()

    def start_store(b, h):
        pltpu.async_copy(out_v.at[b], out_hbm.at[h, :, wid], ssem.at[b])

    def wait_store(b, h):
        pltpu.make_async_copy(out_v.at[b], out_hbm.at[h, :, wid],
                              ssem.at[b]).wait()

    d_lists = [lane + (c * LANES) for c in range(DIM // LANES)]

    def transpose_scale(b):
        def br_body(br, carry):
            cols = jnp.full((LANES,), br, jnp.int32)
            vs = [in_v[b, br, pl.ds(c * LANES, LANES)]
                  for c in range(DIM // LANES)]
            for c, v in enumerate(vs):
                plsc.store_scatter(p_v, [d_lists[c], cols], v)
            return carry

        lax.fori_loop(0, CHUNK, br_body, 0, unroll=2)

        def d_body(d, carry):
            dt = d // DSUB
            dr = d % DSUB
            for c in range(CHUNK // LANES):
                sl = pl.ds(c * LANES, LANES)
                out_v[b, dt, dr, sl] = p_v[d, sl] * SCALE
            return carry

        lax.fori_loop(0, DIM, d_body, 0, unroll=2)

    for b in range(NBUF):
        start_gather(b, b)

    n_blocks = HIST // NBUF

    def block_body(h0, carry):
        for b in range(NBUF):
            h = h0 * NBUF + b
            wait_gather(b, h)

            @pl.when(h0 > 0)
            def _():
                wait_store(b, h - NBUF)

            transpose_scale(b)

            @pl.when(h0 < n_blocks - 1)
            def _():
                start_gather(b, h + NBUF)

            start_store(b, h)
        return carry

    lax.fori_loop(0, n_blocks, block_body, 0)

    for b in range(NBUF):
        wait_store(b, HIST - NBUF + b)


def kernel(x, embedding):
    xt = (x.astype(jnp.int32).T
          .reshape(HIST, NUM_WORKERS, CHUNK)
          .transpose(1, 0, 2))
    r = _embed_sc(xt, embedding)
    return r.transpose(2, 4, 0, 1, 3).reshape(BATCH, HIST, DIM)


# speedup vs baseline: 1.4712x; 1.0012x over previous
import functools

import jax
import jax.numpy as jnp
from jax import lax
from jax.experimental import pallas as pl
from jax.experimental.pallas import tpu as pltpu
from jax.experimental.pallas import tpu_sc as plsc

NUM_CORES = 2
NUM_SUBCORES = 16
NUM_WORKERS = NUM_CORES * NUM_SUBCORES
LANES = 16

BATCH = 4096
HIST = 200
DIM = 64
CHUNK = 128
SCALE = 8.0
NBUF = 5
DSUB = DIM // 8
PSTRIDE = 131

_mesh = plsc.VectorSubcoreMesh(core_axis_name="c", subcore_axis_name="s")


@functools.partial(
    pl.kernel,
    out_type=jax.ShapeDtypeStruct((HIST, 8, NUM_WORKERS, 8, CHUNK),
                                  jnp.float32),
    mesh=_mesh,
    scratch_types=[
        pltpu.VMEM((HIST, CHUNK), jnp.int32),
        pltpu.VMEM((NBUF, CHUNK, DIM), jnp.float32),
        pltpu.VMEM((DIM, PSTRIDE), jnp.float32),
        pltpu.VMEM((NBUF, 8, DSUB, CHUNK), jnp.float32),
        pltpu.SemaphoreType.DMA((NBUF,)),
        pltpu.SemaphoreType.DMA((NBUF,)),
    ],
    compiler_params=pltpu.CompilerParams(use_tc_tiling_on_sc=False,
                                         needs_layout_passes=False),
)
def _embed_sc(idx_hbm, table_hbm, out_hbm, idx_v, in_v, p_v, out_v, gsem, ssem):
    wid = lax.axis_index("s") * NUM_CORES + lax.axis_index("c")
    pltpu.sync_copy(idx_hbm.at[wid], idx_v)
    lane = lax.iota(jnp.int32, LANES)

    def start_gather(b, h):
        pltpu.async_copy(table_hbm.at[idx_v.at[h]], in_v.at[b], gsem.at[b])

    def wait_gather(b, h):
        pltpu.make_async_copy(table_hbm.at[idx_v.at[h]], in_v.at[b],
                              gsem.at[b]).wait()

    def start_store(b, h):
        pltpu.async_copy(out_v.at[b], out_hbm.at[h, :, wid], ssem.at[b])

    def wait_store(b, h):
        pltpu.make_async_copy(out_v.at[b], out_hbm.at[h, :, wid],
                              ssem.at[b]).wait()

    d_lists = [lane + (c * LANES) for c in range(DIM // LANES)]

    def transpose_scale(b):
        def br_body(br, carry):
            cols = jnp.full((LANES,), br, jnp.int32)
            vs = [in_v[b, br, pl.ds(c * LANES, LANES)]
                  for c in range(DIM // LANES)]
            for c, v in enumerate(vs):
                plsc.store_scatter(p_v, [d_lists[c], cols], v)
            return carry

        lax.fori_loop(0, CHUNK, br_body, 0, unroll=2)

        def d_body(d, carry):
            dt = d // DSUB
            dr = d % DSUB
            for c in range(CHUNK // LANES):
                sl = pl.ds(c * LANES, LANES)
                out_v[b, dt, dr, sl] = p_v[d, sl] * SCALE
            return carry

        lax.fori_loop(0, DIM, d_body, 0, unroll=2)

    for b in range(NBUF):
        start_gather(b, b)

    n_blocks = HIST // NBUF

    def block_body(h0, carry):
        for b in range(NBUF):
            h = h0 * NBUF + b
            wait_gather(b, h)

            @pl.when(h0 > 0)
            def _():
                wait_store(b, h - NBUF)

            transpose_scale(b)

            @pl.when(h0 < n_blocks - 1)
            def _():
                start_gather(b, h + NBUF)

            start_store(b, h)
        return carry

    lax.fori_loop(0, n_blocks, block_body, 0)

    for b in range(NBUF):
        wait_store(b, HIST - NBUF + b)


def kernel(x, embedding):
    xt = (x.astype(jnp.int32).T
          .reshape(HIST, NUM_WORKERS, CHUNK)
          .transpose(1, 0, 2))
    r = _embed_sc(xt, embedding)
    return r.transpose(2, 4, 0, 1, 3).reshape(BATCH, HIST, DIM)


# speedup vs baseline: 1.9448x; 1.3219x over previous
import functools

import jax
import jax.numpy as jnp
from jax import lax
from jax.experimental import pallas as pl
from jax.experimental.pallas import tpu as pltpu
from jax.experimental.pallas import tpu_sc as plsc

NUM_CORES = 2
NUM_SUBCORES = 16
NUM_WORKERS = NUM_CORES * NUM_SUBCORES
LANES = 16

BATCH = 4096
HIST = 200
DIM = 64
CHUNK = 128
SCALE = 8.0
NBUF = 5
DSUB = DIM // 8
PSTRIDE = 131

_mesh = plsc.VectorSubcoreMesh(core_axis_name="c", subcore_axis_name="s")


@functools.partial(
    pl.kernel,
    out_type=jax.ShapeDtypeStruct((HIST, 8, NUM_WORKERS, 8, CHUNK),
                                  jnp.float32),
    mesh=_mesh,
    scratch_types=[
        pltpu.VMEM((HIST, CHUNK), jnp.int32),
        pltpu.VMEM((NBUF, CHUNK, DIM), jnp.float32),
        pltpu.VMEM((DIM, PSTRIDE), jnp.float32),
        pltpu.VMEM((NBUF, 8, DSUB, CHUNK), jnp.float32),
        pltpu.SemaphoreType.DMA((NBUF,)),
        pltpu.SemaphoreType.DMA((NBUF,)),
    ],
    compiler_params=pltpu.CompilerParams(use_tc_tiling_on_sc=False,
                                         needs_layout_passes=False),
)
def _embed_sc(idx_hbm, table_hbm, out_hbm, idx_v, in_v, p_v, out_v, gsem, ssem):
    wid = lax.axis_index("s") * NUM_CORES + lax.axis_index("c")
    pltpu.sync_copy(idx_hbm.at[wid], idx_v)
    lane = lax.iota(jnp.int32, LANES)

    def start_gather(b, h):
        pltpu.async_copy(table_hbm.at[idx_v.at[h]], in_v.at[b], gsem.at[b])

    def wait_gather(b, h):
        pltpu.make_async_copy(table_hbm.at[idx_v.at[h]], in_v.at[b],
                              gsem.at[b]).wait()

    def start_store(b, h):
        pltpu.async_copy(out_v.at[b], out_hbm.at[h, :, wid], ssem.at[b])

    def wait_store(b, h):
        pltpu.make_async_copy(out_v.at[b], out_hbm.at[h, :, wid],
                              ssem.at[b]).wait()

    d_lists = [lane + (c * LANES) for c in range(DIM // LANES)]

    def transpose_scale(b):
        def br_body(br, carry):
            cols = jnp.full((LANES,), br, jnp.int32)
            vs = [in_v[b, br, pl.ds(c * LANES, LANES)] * SCALE
                  for c in range(DIM // LANES)]
            for c, v in enumerate(vs):
                plsc.store_scatter(p_v, [d_lists[c], cols], v)
            return carry

        lax.fori_loop(0, CHUNK, br_body, 0, unroll=2)

        def d_body(d, carry):
            dt = d // DSUB
            dr = d % DSUB
            vs = [p_v[d, pl.ds(c * LANES, LANES)]
                  for c in range(CHUNK // LANES)]
            for c, v in enumerate(vs):
                out_v[b, dt, dr, pl.ds(c * LANES, LANES)] = v
            return carry

        lax.fori_loop(0, DIM, d_body, 0, unroll=2)

    for b in range(NBUF):
        start_gather(b, b)

    n_blocks = HIST // NBUF

    def block_body(h0, carry):
        for b in range(NBUF):
            h = h0 * NBUF + b
            wait_gather(b, h)

            @pl.when(h0 > 0)
            def _():
                wait_store(b, h - NBUF)

            transpose_scale(b)

            @pl.when(h0 < n_blocks - 1)
            def _():
                start_gather(b, h + NBUF)

            start_store(b, h)
        return carry

    lax.fori_loop(0, n_blocks, block_body, 0)

    for b in range(NBUF):
        wait_store(b, HIST - NBUF + b)


def kernel(x, embedding):
    xt = (x.astype(jnp.int32).T
          .reshape(HIST, NUM_WORKERS, CHUNK)
          .transpose(1, 0, 2))
    r = _embed_sc(xt, embedding)
    return r.transpose(2, 4, 0, 1, 3).reshape(BATCH, HIST, DIM)


# speedup vs baseline: 1.9520x; 1.0037x over previous
import functools

import jax
import jax.numpy as jnp
from jax import lax
from jax.experimental import pallas as pl
from jax.experimental.pallas import tpu as pltpu
from jax.experimental.pallas import tpu_sc as plsc

NUM_CORES = 2
NUM_SUBCORES = 16
NUM_WORKERS = NUM_CORES * NUM_SUBCORES
LANES = 16

BATCH = 4096
HIST = 200
DIM = 64
CHUNK = 128
SCALE = 8.0
NBUF = 5
DSUB = DIM // 8
PSTRIDE = 131

_mesh = plsc.VectorSubcoreMesh(core_axis_name="c", subcore_axis_name="s")


@functools.partial(
    pl.kernel,
    out_type=jax.ShapeDtypeStruct((HIST, 8, NUM_WORKERS, 8, CHUNK),
                                  jnp.float32),
    mesh=_mesh,
    scratch_types=[
        pltpu.VMEM((HIST, CHUNK), jnp.int32),
        pltpu.VMEM((NBUF, CHUNK, DIM), jnp.float32),
        pltpu.VMEM((DIM, PSTRIDE), jnp.float32),
        pltpu.VMEM((NBUF, 8, DSUB, CHUNK), jnp.float32),
        pltpu.SemaphoreType.DMA((NBUF,)),
        pltpu.SemaphoreType.DMA((NBUF,)),
    ],
    compiler_params=pltpu.CompilerParams(use_tc_tiling_on_sc=False,
                                         needs_layout_passes=False),
)
def _embed_sc(idx_hbm, table_hbm, out_hbm, idx_v, in_v, p_v, out_v, gsem, ssem):
    wid = lax.axis_index("s") * NUM_CORES + lax.axis_index("c")
    pltpu.sync_copy(idx_hbm.at[wid], idx_v)
    lane = lax.iota(jnp.int32, LANES)

    def start_gather(b, h):
        pltpu.async_copy(table_hbm.at[idx_v.at[h]], in_v.at[b], gsem.at[b])

    def wait_gather(b, h):
        pltpu.make_async_copy(table_hbm.at[idx_v.at[h]], in_v.at[b],
                              gsem.at[b]).wait()

    def start_store(b, h):
        pltpu.async_copy(out_v.at[b], out_hbm.at[h, :, wid], ssem.at[b])

    def wait_store(b, h):
        pltpu.make_async_copy(out_v.at[b], out_hbm.at[h, :, wid],
                              ssem.at[b]).wait()

    d_lists = [lane + (c * LANES) for c in range(DIM // LANES)]

    def transpose_scale(b):
        def br_body(br, carry):
            cols = jnp.full((LANES,), br, jnp.int32)
            vs = [in_v[b, br, pl.ds(c * LANES, LANES)] * SCALE
                  for c in range(DIM // LANES)]
            for c, v in enumerate(vs):
                plsc.store_scatter(p_v, [d_lists[c], cols], v)
            return carry

        lax.fori_loop(0, CHUNK, br_body, 0, unroll=4)

        def d_body(d, carry):
            dt = d // DSUB
            dr = d % DSUB
            vs = [p_v[d, pl.ds(c * LANES, LANES)]
                  for c in range(CHUNK // LANES)]
            for c, v in enumerate(vs):
                out_v[b, dt, dr, pl.ds(c * LANES, LANES)] = v
            return carry

        lax.fori_loop(0, DIM, d_body, 0, unroll=4)

    for b in range(NBUF):
        start_gather(b, b)

    n_blocks = HIST // NBUF

    def block_body(h0, carry):
        for b in range(NBUF):
            h = h0 * NBUF + b
            wait_gather(b, h)

            @pl.when(h0 > 0)
            def _():
                wait_store(b, h - NBUF)

            transpose_scale(b)

            @pl.when(h0 < n_blocks - 1)
            def _():
                start_gather(b, h + NBUF)

            start_store(b, h)
        return carry

    lax.fori_loop(0, n_blocks, block_body, 0)

    for b in range(NBUF):
        wait_store(b, HIST - NBUF + b)


def kernel(x, embedding):
    xt = (x.astype(jnp.int32).T
          .reshape(HIST, NUM_WORKERS, CHUNK)
          .transpose(1, 0, 2))
    r = _embed_sc(xt, embedding)
    return r.transpose(2, 4, 0, 1, 3).reshape(BATCH, HIST, DIM)
